# baseline (speedup 1.0000x reference)
import functools

import jax
import jax.numpy as jnp
from jax import lax
from jax.experimental import pallas as pl
from jax.experimental.pallas import tpu as pltpu
from jax.experimental.pallas import tpu_sc as plsc

_D = 128
_T = 4
_STEPS = 6
_NC = 2
_NS = 16
_NW = _NC * _NS
_CHUNK = 128


def _transform(h, W_e, b_e, n):
    bn = 2000

    def body(h_ref, w_ref, b_ref, o_ref):
        hb = h_ref[...]
        for t in range(_T):
            o_ref[t] = lax.dot_general(
                hb, w_ref[t], (((1,), (1,)), ((), ())),
                preferred_element_type=jnp.float32) + b_ref[t][None, :]

    return pl.pallas_call(
        body,
        grid=(n // bn,),
        in_specs=[
            pl.BlockSpec((bn, _D), lambda i: (i, 0)),
            pl.BlockSpec((_T, _D, _D), lambda i: (0, 0, 0)),
            pl.BlockSpec((_T, _D), lambda i: (0, 0)),
        ],
        out_specs=pl.BlockSpec((_T, bn, _D), lambda i: (0, i, 0)),
        out_shape=jax.ShapeDtypeStruct((_T, n, _D), jnp.float32),
    )(h, W_e, b_e)


def _gru(parts, h, w_ih, w_hh, b_ih, b_hh, n, W_e=None, b_e=None,
         feature=None):
    bn = 2000

    def gru_block(a0_ref, a1_ref, h_ref, wih_ref, whh_ref, bih_ref,
                  bhh_ref):
        a = a0_ref[0] + a1_ref[0]
        hb = h_ref[...]
        gi = lax.dot_general(a, wih_ref[...], (((1,), (1,)), ((), ())),
                             preferred_element_type=jnp.float32)
        gi = gi + bih_ref[...][None, :]
        gh = lax.dot_general(hb, whh_ref[...], (((1,), (1,)), ((), ())),
                             preferred_element_type=jnp.float32)
        gh = gh + bhh_ref[...][None, :]
        r = jax.nn.sigmoid(gi[:, :_D] + gh[:, :_D])
        z = jax.nn.sigmoid(gi[:, _D:2 * _D] + gh[:, _D:2 * _D])
        nn = jnp.tanh(gi[:, 2 * _D:] + r * gh[:, 2 * _D:])
        return (1.0 - z) * nn + z * hb

    base_specs = [
        pl.BlockSpec((1, bn, _D), lambda i: (0, i, 0)),
        pl.BlockSpec((1, bn, _D), lambda i: (1, i, 0)),
        pl.BlockSpec((bn, _D), lambda i: (i, 0)),
        pl.BlockSpec((3 * _D, _D), lambda i: (0, 0)),
        pl.BlockSpec((3 * _D, _D), lambda i: (0, 0)),
        pl.BlockSpec((3 * _D,), lambda i: (0,)),
        pl.BlockSpec((3 * _D,), lambda i: (0,)),
    ]

    if W_e is not None:
        def body(a0_ref, a1_ref, h_ref, wih_ref, whh_ref, bih_ref,
                 bhh_ref, we_ref, be_ref, hn_ref, xt_ref):
            hn = gru_block(a0_ref, a1_ref, h_ref, wih_ref, whh_ref,
                           bih_ref, bhh_ref)
            hn_ref[...] = hn
            for t in range(_T):
                xt_ref[t] = lax.dot_general(
                    hn, we_ref[t], (((1,), (1,)), ((), ())),
                    preferred_element_type=jnp.float32) + be_ref[t][None, :]

        return pl.pallas_call(
            body,
            grid=(n // bn,),
            in_specs=base_specs + [
                pl.BlockSpec((_T, _D, _D), lambda i: (0, 0, 0)),
                pl.BlockSpec((_T, _D), lambda i: (0, 0)),
            ],
            out_specs=[
                pl.BlockSpec((bn, _D), lambda i: (i, 0)),
                pl.BlockSpec((_T, bn, _D), lambda i: (0, i, 0)),
            ],
            out_shape=[
                jax.ShapeDtypeStruct((n, _D), jnp.float32),
                jax.ShapeDtypeStruct((_T, n, _D), jnp.float32),
            ],
        )(parts, parts, h, w_ih, w_hh, b_ih, b_hh, W_e, b_e)

    def body(a0_ref, a1_ref, h_ref, wih_ref, whh_ref, bih_ref,
             bhh_ref, f_ref, out_ref):
        hn = gru_block(a0_ref, a1_ref, h_ref, wih_ref, whh_ref,
                       bih_ref, bhh_ref)
        out_ref[:, :_D] = hn
        out_ref[:, _D:] = f_ref[...]

    return pl.pallas_call(
        body,
        grid=(n // bn,),
        in_specs=base_specs + [pl.BlockSpec((bn, _D), lambda i: (i, 0))],
        out_specs=pl.BlockSpec((bn, 2 * _D), lambda i: (i, 0)),
        out_shape=jax.ShapeDtypeStruct((n, 2 * _D), jnp.float32),
    )(parts, parts, h, w_ih, w_hh, b_ih, b_hh, feature)


def _sc_aggregate(xt_flat, gidx, dsts, zeros, n, cpw0, cpw1, acc_rows):
    zrows = acc_rows // _NS
    half = max(cpw0, cpw1) // 2
    mesh = plsc.VectorSubcoreMesh(core_axis_name="c", subcore_axis_name="s")

    @functools.partial(
        pl.kernel,
        out_type=jax.ShapeDtypeStruct((_NC, acc_rows, _D), jnp.float32),
        mesh=mesh,
        scratch_types=[
            pltpu.VMEM((half, _CHUNK), jnp.int32),
            pltpu.VMEM((half, _CHUNK), jnp.int32),
            pltpu.VMEM((_CHUNK, _D), jnp.float32),
            pltpu.VMEM((_CHUNK, _D), jnp.float32),
            pltpu.VMEM_SHARED((acc_rows, _D), jnp.float32),
            pltpu.SemaphoreType.DMA,
            pltpu.SemaphoreType.DMA,
        ],
    )
    def k(xt_hbm, gidx_hbm, dst_hbm, zeros_hbm, out_hbm,
          idx_v, dst_v, rows0, rows1, acc, sem0, sem1):
        c = lax.axis_index("c")
        s = lax.axis_index("s")
        wid = c * _NS + s
        nh = jnp.where(c == 0, cpw0 // 4, cpw1 // 4)
        lim = 2 * nh
        pltpu.sync_copy(zeros_hbm.at[pl.ds(s * zrows, zrows)],
                        acc.at[pl.ds(s * zrows, zrows)])
        plsc.subcore_barrier()

        for hh in range(2):
            pltpu.sync_copy(gidx_hbm.at[wid, hh], idx_v)
            pltpu.sync_copy(dst_hbm.at[wid, hh], dst_v)
            pltpu.async_copy(xt_hbm.at[idx_v.at[0]], rows0, sem0)

            def body(i, carry):
                j = 2 * i
                pltpu.async_copy(xt_hbm.at[idx_v.at[j + 1]], rows1, sem1)
                pltpu.make_async_copy(
                    xt_hbm.at[idx_v.at[j]], rows0, sem0).wait()
                pltpu.sync_copy(rows0, acc.at[dst_v.at[j]], add=True)

                @pl.when(j + 2 < lim)
                def _():
                    pltpu.async_copy(xt_hbm.at[idx_v.at[j + 2]], rows0,
                                     sem0)

                pltpu.make_async_copy(
                    xt_hbm.at[idx_v.at[j + 1]], rows1, sem1).wait()
                pltpu.sync_copy(rows1, acc.at[dst_v.at[j + 1]], add=True)
                return carry

            lax.fori_loop(0, nh, body, 0)
        plsc.subcore_barrier()
        pltpu.sync_copy(acc.at[pl.ds(s * zrows, zrows)],
                        out_hbm.at[c, pl.ds(s * zrows, zrows)])

    return k(xt_flat, gidx, dsts, zeros)


def kernel(feature, edge_index, etypes, W_e, b_e, w_ih, w_hh, b_ih, b_hh):
    n = feature.shape[0]
    e = edge_index.shape[1]
    cpw1 = min(-(-int(e * 0.5) // (_NS * _CHUNK * 4)) * 4,
               -(-e // (_NS * _CHUNK * 4)) * 4)
    rem = e - _NS * cpw1 * _CHUNK
    cpw0 = max(-(-rem // (_NS * _CHUNK * 4)) * 4, 4)
    acc_rows = -(-n // (_NS * 8)) * (_NS * 8)
    if acc_rows == n:
        acc_rows += _NS * 8

    src = edge_index[0]
    dst = edge_index[1]
    cap = max(cpw0, cpw1)
    epw = -(-e // _NW)
    tail = _NW * epw - e
    padw = cap * _CHUNK - epw

    def layout(x, spread):
        x = jnp.concatenate(
            [x, jnp.full((tail,), n if not spread else 0, jnp.int32)])
        x = x.reshape(_NW, epw)
        if spread:
            fillrow = (jnp.arange(padw, dtype=jnp.int32) * 997) % (_T * n)
        else:
            fillrow = n + jnp.arange(padw, dtype=jnp.int32) % (acc_rows - n)
        fill2 = jnp.broadcast_to(fillrow[None, :], (_NW, padw))
        return jnp.concatenate([x, fill2], axis=1).reshape(
            _NW, 2, cap // 2, _CHUNK)

    gidx = layout(etypes * n + src, True)
    dsts = layout(dst, False)
    zeros = jnp.zeros((acc_rows, _D), jnp.float32)

    h = feature
    xt = _transform(h, W_e, b_e, n)
    for _step in range(_STEPS):
        parts = _sc_aggregate(xt.reshape(_T * n, _D), gidx, dsts, zeros,
                              n, cpw0, cpw1, acc_rows)
        if _step < _STEPS - 1:
            h, xt = _gru(parts, h, w_ih, w_hh, b_ih, b_hh, n,
                         W_e=W_e, b_e=b_e)
        else:
            out = _gru(parts, h, w_ih, w_hh, b_ih, b_hh, n,
                       feature=feature)
    return out


# speedup vs baseline: 1.0026x; 1.0026x over previous
import functools

import jax
import jax.numpy as jnp
from jax import lax
from jax.experimental import pallas as pl
from jax.experimental.pallas import tpu as pltpu
from jax.experimental.pallas import tpu_sc as plsc

_D = 128
_T = 4
_STEPS = 6
_NC = 2
_NS = 16
_NW = _NC * _NS
_CHUNK = 128


def _transform(h, W_e, b_e, n):
    bn = 2000

    def body(h_ref, w_ref, b_ref, o_ref):
        hb = h_ref[...]
        for t in range(_T):
            o_ref[t] = lax.dot_general(
                hb, w_ref[t], (((1,), (1,)), ((), ())),
                preferred_element_type=jnp.float32) + b_ref[t][None, :]

    return pl.pallas_call(
        body,
        grid=(n // bn,),
        in_specs=[
            pl.BlockSpec((bn, _D), lambda i: (i, 0)),
            pl.BlockSpec((_T, _D, _D), lambda i: (0, 0, 0)),
            pl.BlockSpec((_T, _D), lambda i: (0, 0)),
        ],
        out_specs=pl.BlockSpec((_T, bn, _D), lambda i: (0, i, 0)),
        out_shape=jax.ShapeDtypeStruct((_T, n, _D), jnp.float32),
    )(h, W_e, b_e)


def _gru(parts, h, w_ih, w_hh, b_ih, b_hh, n, W_e=None, b_e=None,
         feature=None):
    bn = 2000

    def gru_block(a0_ref, a1_ref, h_ref, wih_ref, whh_ref, bih_ref,
                  bhh_ref):
        a = a0_ref[0] + a1_ref[0]
        hb = h_ref[...]
        gi = lax.dot_general(a, wih_ref[...], (((1,), (1,)), ((), ())),
                             preferred_element_type=jnp.float32)
        gi = gi + bih_ref[...][None, :]
        gh = lax.dot_general(hb, whh_ref[...], (((1,), (1,)), ((), ())),
                             preferred_element_type=jnp.float32)
        gh = gh + bhh_ref[...][None, :]
        r = jax.nn.sigmoid(gi[:, :_D] + gh[:, :_D])
        z = jax.nn.sigmoid(gi[:, _D:2 * _D] + gh[:, _D:2 * _D])
        nn = jnp.tanh(gi[:, 2 * _D:] + r * gh[:, 2 * _D:])
        return (1.0 - z) * nn + z * hb

    base_specs = [
        pl.BlockSpec((1, bn, _D), lambda i: (0, i, 0)),
        pl.BlockSpec((1, bn, _D), lambda i: (1, i, 0)),
        pl.BlockSpec((bn, _D), lambda i: (i, 0)),
        pl.BlockSpec((3 * _D, _D), lambda i: (0, 0)),
        pl.BlockSpec((3 * _D, _D), lambda i: (0, 0)),
        pl.BlockSpec((3 * _D,), lambda i: (0,)),
        pl.BlockSpec((3 * _D,), lambda i: (0,)),
    ]

    if W_e is not None:
        def body(a0_ref, a1_ref, h_ref, wih_ref, whh_ref, bih_ref,
                 bhh_ref, we_ref, be_ref, hn_ref, xt_ref):
            hn = gru_block(a0_ref, a1_ref, h_ref, wih_ref, whh_ref,
                           bih_ref, bhh_ref)
            hn_ref[...] = hn
            for t in range(_T):
                xt_ref[t] = lax.dot_general(
                    hn, we_ref[t], (((1,), (1,)), ((), ())),
                    preferred_element_type=jnp.float32) + be_ref[t][None, :]

        return pl.pallas_call(
            body,
            grid=(n // bn,),
            in_specs=base_specs + [
                pl.BlockSpec((_T, _D, _D), lambda i: (0, 0, 0)),
                pl.BlockSpec((_T, _D), lambda i: (0, 0)),
            ],
            out_specs=[
                pl.BlockSpec((bn, _D), lambda i: (i, 0)),
                pl.BlockSpec((_T, bn, _D), lambda i: (0, i, 0)),
            ],
            out_shape=[
                jax.ShapeDtypeStruct((n, _D), jnp.float32),
                jax.ShapeDtypeStruct((_T, n, _D), jnp.float32),
            ],
        )(parts, parts, h, w_ih, w_hh, b_ih, b_hh, W_e, b_e)

    def body(a0_ref, a1_ref, h_ref, wih_ref, whh_ref, bih_ref,
             bhh_ref, f_ref, out_ref):
        hn = gru_block(a0_ref, a1_ref, h_ref, wih_ref, whh_ref,
                       bih_ref, bhh_ref)
        out_ref[:, :_D] = hn
        out_ref[:, _D:] = f_ref[...]

    return pl.pallas_call(
        body,
        grid=(n // bn,),
        in_specs=base_specs + [pl.BlockSpec((bn, _D), lambda i: (i, 0))],
        out_specs=pl.BlockSpec((bn, 2 * _D), lambda i: (i, 0)),
        out_shape=jax.ShapeDtypeStruct((n, 2 * _D), jnp.float32),
    )(parts, parts, h, w_ih, w_hh, b_ih, b_hh, feature)


def _sc_aggregate(xt_flat, gidx, dsts, zeros, n, cpw0, cpw1, acc_rows):
    zrows = acc_rows // _NS
    half = max(cpw0, cpw1) // 2
    mesh = plsc.VectorSubcoreMesh(core_axis_name="c", subcore_axis_name="s")

    @functools.partial(
        pl.kernel,
        out_type=jax.ShapeDtypeStruct((_NC, acc_rows, _D), jnp.float32),
        mesh=mesh,
        scratch_types=[
            pltpu.VMEM((half, _CHUNK), jnp.int32),
            pltpu.VMEM((half, _CHUNK), jnp.int32),
            pltpu.VMEM((_CHUNK, _D), jnp.float32),
            pltpu.VMEM((_CHUNK, _D), jnp.float32),
            pltpu.VMEM_SHARED((acc_rows, _D), jnp.float32),
            pltpu.SemaphoreType.DMA,
            pltpu.SemaphoreType.DMA,
        ],
    )
    def k(xt_hbm, gidx_hbm, dst_hbm, zeros_hbm, out_hbm,
          idx_v, dst_v, rows0, rows1, acc, sem0, sem1):
        c = lax.axis_index("c")
        s = lax.axis_index("s")
        wid = c * _NS + s
        nh = jnp.where(c == 0, cpw0 // 4, cpw1 // 4)
        lim = 2 * nh
        pltpu.sync_copy(zeros_hbm.at[pl.ds(s * zrows, zrows)],
                        acc.at[pl.ds(s * zrows, zrows)])
        plsc.subcore_barrier()

        for hh in range(2):
            pltpu.sync_copy(gidx_hbm.at[wid, hh], idx_v)
            pltpu.sync_copy(dst_hbm.at[wid, hh], dst_v)
            pltpu.async_copy(xt_hbm.at[idx_v.at[0]], rows0, sem0)

            def body(i, carry):
                j = 2 * i
                pltpu.async_copy(xt_hbm.at[idx_v.at[j + 1]], rows1, sem1)
                pltpu.make_async_copy(
                    xt_hbm.at[idx_v.at[j]], rows0, sem0).wait()
                pltpu.sync_copy(rows0, acc.at[dst_v.at[j]], add=True)

                @pl.when(j + 2 < lim)
                def _():
                    pltpu.async_copy(xt_hbm.at[idx_v.at[j + 2]], rows0,
                                     sem0)

                pltpu.make_async_copy(
                    xt_hbm.at[idx_v.at[j + 1]], rows1, sem1).wait()
                pltpu.sync_copy(rows1, acc.at[dst_v.at[j + 1]], add=True)
                return carry

            lax.fori_loop(0, nh, body, 0)
        plsc.subcore_barrier()
        pltpu.sync_copy(acc.at[pl.ds(s * zrows, zrows)],
                        out_hbm.at[c, pl.ds(s * zrows, zrows)])

    return k(xt_flat, gidx, dsts, zeros)


def kernel(feature, edge_index, etypes, W_e, b_e, w_ih, w_hh, b_ih, b_hh):
    n = feature.shape[0]
    e = edge_index.shape[1]
    cpw0 = cpw1 = -(-e // (_NW * _CHUNK * 4)) * 4
    acc_rows = -(-n // (_NS * 8)) * (_NS * 8)
    if acc_rows == n:
        acc_rows += _NS * 8

    src = edge_index[0]
    dst = edge_index[1]
    cap = max(cpw0, cpw1)
    epw = -(-e // _NW)
    tail = _NW * epw - e
    padw = cap * _CHUNK - epw

    def layout(x, spread):
        x = jnp.concatenate(
            [x, jnp.full((tail,), n if not spread else 0, jnp.int32)])
        x = x.reshape(_NW, epw)
        if spread:
            fillrow = (jnp.arange(padw, dtype=jnp.int32) * 997) % (_T * n)
        else:
            fillrow = n + jnp.arange(padw, dtype=jnp.int32) % (acc_rows - n)
        fill2 = jnp.broadcast_to(fillrow[None, :], (_NW, padw))
        return jnp.concatenate([x, fill2], axis=1).reshape(
            _NW, 2, cap // 2, _CHUNK)

    gidx = layout(etypes * n + src, True)
    dsts = layout(dst, False)
    zeros = jnp.zeros((acc_rows, _D), jnp.float32)

    h = feature
    xt = _transform(h, W_e, b_e, n)
    for _step in range(_STEPS):
        parts = _sc_aggregate(xt.reshape(_T * n, _D), gidx, dsts, zeros,
                              n, cpw0, cpw1, acc_rows)
        if _step < _STEPS - 1:
            h, xt = _gru(parts, h, w_ih, w_hh, b_ih, b_hh, n,
                         W_e=W_e, b_e=b_e)
        else:
            out = _gru(parts, h, w_ih, w_hh, b_ih, b_hh, n,
                       feature=feature)
    return out
